# baseline (speedup 1.0000x reference)
import functools

import numpy as np
import jax
import jax.numpy as jnp
from jax import lax
from jax.experimental import pallas as pl
from jax.experimental.pallas import tpu as pltpu
from jax.experimental.pallas import tpu_sc as plsc

_B, _H, _W = 16, 480, 640
_GROUPS_PER_TEC = (_H // 2) // 16
_WORDS = _H * _W // 8

_WMAT = np.zeros((16, 8), np.float32)
_NSEL = np.zeros((16, 8), np.float32)
for _slot, (_J, _Ax, _Bx, _Ay, _By) in enumerate([
    (2.0, 225.0, 50625.0, 128.0, 16384.0),
    (1.0, 0.0, 0.0, 0.0, 0.0),
    (2.0, 0.0, 0.0, 0.0, 0.0),
]):
    _WMAT[5 * _slot + 0, _slot] = _Bx + _By
    _WMAT[5 * _slot + 1, _slot] = -2.0 * _Ax
    _WMAT[5 * _slot + 2, _slot] = _J
    _WMAT[5 * _slot + 3, _slot] = -2.0 * _Ay
    _WMAT[5 * _slot + 4, _slot] = _J
    _NSEL[5 * _slot + 0, _slot] = 1.0


def _pack_body(lab_ref, out_ref):
    w = lab_ref[0, :, 0, :]
    for j in range(1, 8):
        w = w | (lab_ref[0, :, j, :] << (4 * j))
    out_ref[0] = w


_pack_call = pl.pallas_call(
    _pack_body,
    grid=(_B,),
    in_specs=[pl.BlockSpec((1, _H // 8, 8, _W), lambda b: (b, 0, 0, 0))],
    out_specs=pl.BlockSpec((1, _H // 8, _W), lambda b: (b, 0, 0)),
    out_shape=jax.ShapeDtypeStruct((_B, _H // 8, _W), jnp.int32),
)


_sc_mesh = plsc.VectorSubcoreMesh(core_axis_name="c", subcore_axis_name="s")


@functools.partial(
    pl.kernel,
    mesh=_sc_mesh,
    compiler_params=pltpu.CompilerParams(needs_layout_passes=False),
    out_type=jax.ShapeDtypeStruct((32, 15, 16), jnp.float32),
    scratch_types=[
        pltpu.VMEM((_WORDS,), jnp.int32),
        pltpu.VMEM((9, 16), jnp.float32),
        pltpu.VMEM((15, 16), jnp.float32),
        pltpu.VMEM((16, 16), jnp.float32),
        pltpu.VMEM((16, 16), jnp.float32),
        pltpu.VMEM((16, 16), jnp.float32),
    ],
)
def _sc_moments(packed_hbm, hb_hbm, out_hbm, table_v, h_v, mom_v,
                hist_n, hist_c, hist_c2):
    cid = lax.axis_index("c")
    sid = lax.axis_index("s")
    wid = sid * 2 + cid
    batch = wid >> 1
    half = wid & 1

    pltpu.sync_copy(packed_hbm.at[batch], table_v)
    pltpu.sync_copy(hb_hbm.at[batch], h_v)

    h00 = h_v[0]
    h01 = h_v[1]
    h02 = h_v[2]
    h10 = h_v[3]
    h11 = h_v[4]
    h12 = h_v[5]
    h20 = h_v[6]
    h21 = h_v[7]
    h22 = h_v[8]

    iota16 = lax.iota(jnp.int32, 16)
    iotaf = iota16.astype(jnp.float32)
    z = jnp.zeros((16,), jnp.float32)
    ones = jnp.ones((16,), jnp.float32)
    for i in range(16):
        hist_n[i] = z
        hist_c[i] = z
        hist_c2[i] = z

    accs = [z] * 9

    r_base = (half * (_H // 2)).astype(jnp.float32)
    for g in range(_GROUPS_PER_TEC):
        rvf = r_base + (g * 16) + iotaf
        bx = h01 * rvf + h02
        by = h11 * rvf + h12
        bd = h21 * rvf + h22

        def body(ci, carry, bx=bx, by=by, bd=bd):
            cfv = ci.astype(jnp.float32) + z
            den = h20 * cfv + bd
            rcp = 1.0 / den
            xs = (h00 * cfv + bx) * rcp
            ys = (h10 * cfv + by) * rcp
            xi = (jnp.clip(xs, 0.0, float(_W - 1)) + 0.5).astype(jnp.int32)
            yi = (jnp.clip(ys, 0.0, float(_H - 1)) + 0.5).astype(jnp.int32)
            lin = (yi >> 3) * _W + xi
            shift = (yi & 7) << 2
            w = plsc.load_gather(table_v, [lin])
            code = (w >> shift) & 15
            plsc.addupdate_scatter(hist_n, [iota16, code], ones)
            plsc.addupdate_scatter(hist_c, [iota16, code], cfv)
            plsc.addupdate_scatter(hist_c2, [iota16, code], cfv * cfv)
            return carry

        lax.fori_loop(0, _W, body, 0, unroll=4)

        rvf2 = rvf * rvf
        for slot, k in enumerate((3, 4, 8)):
            kidx = jnp.full((16,), k, jnp.int32)
            cnt = plsc.load_gather(hist_n, [iota16, kidx])
            plsc.addupdate_scatter(hist_n, [iota16, kidx], -cnt)
            accs[3 * slot + 0] = accs[3 * slot + 0] + cnt
            accs[3 * slot + 1] = accs[3 * slot + 1] + rvf * cnt
            accs[3 * slot + 2] = accs[3 * slot + 2] + rvf2 * cnt

    for slot, k in enumerate((3, 4, 8)):
        kidx = jnp.full((16,), k, jnp.int32)
        mom_v[5 * slot + 0] = accs[3 * slot + 0]
        mom_v[5 * slot + 1] = accs[3 * slot + 1]
        mom_v[5 * slot + 2] = accs[3 * slot + 2]
        mom_v[5 * slot + 3] = plsc.load_gather(hist_c, [iota16, kidx])
        mom_v[5 * slot + 4] = plsc.load_gather(hist_c2, [iota16, kidx])
    pltpu.sync_copy(mom_v, out_hbm.at[wid])


def _loss_body(m_ref, w_ref, ns_ref, out_ref):
    s = (m_ref[:, 0] + m_ref[:, 1]).sum(axis=2)
    lin = jnp.dot(s, w_ref[...], preferred_element_type=jnp.float32)
    nsel = jnp.dot(s, ns_ref[...], preferred_element_type=jnp.float32)
    out_ref[...] = jnp.sum(lin / jnp.maximum(1.0, nsel)).reshape(1, 1)


_loss_call = pl.pallas_call(
    _loss_body,
    out_shape=jax.ShapeDtypeStruct((1, 1), jnp.float32),
)


def kernel(input_label, h, device=0):
    lab4 = input_label.reshape(_B, _H // 8, 8, _W)
    packed = _pack_call(lab4).reshape(_B, _WORDS)
    hb = jnp.broadcast_to(
        h.astype(jnp.float32).reshape(_B, 9, 1), (_B, 9, 16))
    mom = _sc_moments(packed, hb)
    m4d = mom.reshape(_B, 2, 15, 16)
    m4d = jnp.pad(m4d, ((0, 0), (0, 0), (0, 1), (0, 0)))
    loss = _loss_call(m4d, jnp.asarray(_WMAT), jnp.asarray(_NSEL))
    return loss[0, 0]


# speedup vs baseline: 2.0922x; 2.0922x over previous
import functools

import numpy as np
import jax
import jax.numpy as jnp
from jax import lax
from jax.experimental import pallas as pl
from jax.experimental.pallas import tpu as pltpu
from jax.experimental.pallas import tpu_sc as plsc

_B, _H, _W = 16, 480, 640
_GROUPS_PER_TEC = (_H // 2) // 16
_WORDS = _H * _W // 8

_WMAT = np.zeros((16, 8), np.float32)
_NSEL = np.zeros((16, 8), np.float32)
for _slot, (_J, _Ax, _Bx, _Ay, _By) in enumerate([
    (2.0, 225.0, 50625.0, 128.0, 16384.0),
    (1.0, 0.0, 0.0, 0.0, 0.0),
    (2.0, 0.0, 0.0, 0.0, 0.0),
]):
    _WMAT[5 * _slot + 0, _slot] = _Bx + _By
    _WMAT[5 * _slot + 1, _slot] = -2.0 * _Ax
    _WMAT[5 * _slot + 2, _slot] = _J
    _WMAT[5 * _slot + 3, _slot] = -2.0 * _Ay
    _WMAT[5 * _slot + 4, _slot] = _J
    _NSEL[5 * _slot + 0, _slot] = 1.0


def _pack_body(lab_ref, out_ref):
    w = lab_ref[0, :, 0, :]
    for j in range(1, 8):
        w = w | (lab_ref[0, :, j, :] << (4 * j))
    out_ref[0] = w


_pack_call = pl.pallas_call(
    _pack_body,
    grid=(_B,),
    in_specs=[pl.BlockSpec((1, _H // 8, 8, _W), lambda b: (b, 0, 0, 0))],
    out_specs=pl.BlockSpec((1, _H // 8, _W), lambda b: (b, 0, 0)),
    out_shape=jax.ShapeDtypeStruct((_B, _H // 8, _W), jnp.int32),
)


_sc_mesh = plsc.VectorSubcoreMesh(core_axis_name="c", subcore_axis_name="s")


@functools.partial(
    pl.kernel,
    mesh=_sc_mesh,
    compiler_params=pltpu.CompilerParams(needs_layout_passes=False),
    out_type=jax.ShapeDtypeStruct((32, 15, 16), jnp.float32),
    scratch_types=[
        pltpu.VMEM((_WORDS,), jnp.int32),
        pltpu.VMEM((9, 16), jnp.float32),
        pltpu.VMEM((15, 16), jnp.float32),
        pltpu.VMEM((16, 16), jnp.float32),
        pltpu.VMEM((16, 16), jnp.float32),
        pltpu.VMEM((16, 16), jnp.float32),
    ],
)
def _sc_moments(packed_hbm, hb_hbm, out_hbm, table_v, h_v, mom_v,
                hist_n, hist_c, hist_c2):
    cid = lax.axis_index("c")
    sid = lax.axis_index("s")
    wid = sid * 2 + cid
    batch = wid >> 1
    half = wid & 1

    pltpu.sync_copy(packed_hbm.at[batch], table_v)
    pltpu.sync_copy(hb_hbm.at[batch], h_v)

    h00 = h_v[0]
    h01 = h_v[1]
    h02 = h_v[2]
    h10 = h_v[3]
    h11 = h_v[4]
    h12 = h_v[5]
    h20 = h_v[6]
    h21 = h_v[7]
    h22 = h_v[8]

    iota16 = lax.iota(jnp.int32, 16)
    iotaf = iota16.astype(jnp.float32)
    z = jnp.zeros((16,), jnp.float32)

    accs = [z] * 15

    r_base = (half * (_H // 2)).astype(jnp.float32)
    for g in range(_GROUPS_PER_TEC):
        rvf = r_base + (g * 16) + iotaf
        bx = h01 * rvf + h02
        by = h11 * rvf + h12
        bd = h21 * rvf + h22

        def body(ci, carry):
            c3, s3, q3, c4, s4, q4, c8, s8, q8, nx, ny, dn, cfv = carry
            rcp = 1.0 / dn
            xs = nx * rcp
            ys = ny * rcp
            xi = (jnp.clip(xs, 0.0, float(_W - 1)) + 0.5).astype(jnp.int32)
            yi = (jnp.clip(ys, 0.0, float(_H - 1)) + 0.5).astype(jnp.int32)
            lin = (yi >> 3) * _W + xi
            shift = (yi & 7) << 2
            w = plsc.load_gather(table_v, [lin])
            code = (w >> shift) & 15
            m3 = jnp.where(code == 3, 1.0, 0.0)
            m4 = jnp.where(code == 4, 1.0, 0.0)
            m8 = jnp.where(code == 8, 1.0, 0.0)
            cf2 = cfv * cfv
            return (c3 + m3, s3 + cfv * m3, q3 + cf2 * m3,
                    c4 + m4, s4 + cfv * m4, q4 + cf2 * m4,
                    c8 + m8, s8 + cfv * m8, q8 + cf2 * m8,
                    nx + h00, ny + h10, dn + h20, cfv + 1.0)

        init = (z, z, z, z, z, z, z, z, z, bx, by, bd, z)
        out = lax.fori_loop(0, _W, body, init, unroll=8)
        c3, s3, q3, c4, s4, q4, c8, s8, q8 = out[:9]

        rvf2 = rvf * rvf
        for slot, (cnt, sc_, sc2_) in enumerate(
                ((c3, s3, q3), (c4, s4, q4), (c8, s8, q8))):
            accs[5 * slot + 0] = accs[5 * slot + 0] + cnt
            accs[5 * slot + 1] = accs[5 * slot + 1] + rvf * cnt
            accs[5 * slot + 2] = accs[5 * slot + 2] + rvf2 * cnt
            accs[5 * slot + 3] = accs[5 * slot + 3] + sc_
            accs[5 * slot + 4] = accs[5 * slot + 4] + sc2_

    for i in range(15):
        mom_v[i] = accs[i]
    pltpu.sync_copy(mom_v, out_hbm.at[wid])


def _loss_body(m_ref, w_ref, ns_ref, out_ref):
    s = (m_ref[:, 0] + m_ref[:, 1]).sum(axis=2)
    lin = jnp.dot(s, w_ref[...], preferred_element_type=jnp.float32)
    nsel = jnp.dot(s, ns_ref[...], preferred_element_type=jnp.float32)
    out_ref[...] = jnp.sum(lin / jnp.maximum(1.0, nsel)).reshape(1, 1)


_loss_call = pl.pallas_call(
    _loss_body,
    out_shape=jax.ShapeDtypeStruct((1, 1), jnp.float32),
)


def kernel(input_label, h, device=0):
    lab4 = input_label.reshape(_B, _H // 8, 8, _W)
    packed = _pack_call(lab4).reshape(_B, _WORDS)
    hb = jnp.broadcast_to(
        h.astype(jnp.float32).reshape(_B, 9, 1), (_B, 9, 16))
    mom = _sc_moments(packed, hb)
    m4d = mom.reshape(_B, 2, 15, 16)
    m4d = jnp.pad(m4d, ((0, 0), (0, 0), (0, 1), (0, 0)))
    loss = _loss_call(m4d, jnp.asarray(_WMAT), jnp.asarray(_NSEL))
    return loss[0, 0]


# speedup vs baseline: 2.3749x; 1.1351x over previous
import functools

import numpy as np
import jax
import jax.numpy as jnp
from jax import lax
from jax.experimental import pallas as pl
from jax.experimental.pallas import tpu as pltpu
from jax.experimental.pallas import tpu_sc as plsc

_B, _H, _W = 16, 480, 640
_GROUPS_PER_TEC = (_H // 2) // 16
_WORDS = _H * _W // 8

_WMAT = np.zeros((16, 8), np.float32)
_NSEL = np.zeros((16, 8), np.float32)
for _slot, (_J, _Ax, _Bx, _Ay, _By) in enumerate([
    (2.0, 225.0, 50625.0, 128.0, 16384.0),
    (1.0, 0.0, 0.0, 0.0, 0.0),
    (2.0, 0.0, 0.0, 0.0, 0.0),
]):
    _WMAT[5 * _slot + 0, _slot] = _Bx + _By
    _WMAT[5 * _slot + 1, _slot] = -2.0 * _Ax
    _WMAT[5 * _slot + 2, _slot] = _J
    _WMAT[5 * _slot + 3, _slot] = -2.0 * _Ay
    _WMAT[5 * _slot + 4, _slot] = _J
    _NSEL[5 * _slot + 0, _slot] = 1.0


def _pack_body(lab_ref, out_ref):
    w = lab_ref[0, :, 0, :]
    for j in range(1, 8):
        w = w | (lab_ref[0, :, j, :] << (4 * j))
    out_ref[0] = w


_pack_call = pl.pallas_call(
    _pack_body,
    grid=(_B,),
    in_specs=[pl.BlockSpec((1, _H // 8, 8, _W), lambda b: (b, 0, 0, 0))],
    out_specs=pl.BlockSpec((1, _H // 8, _W), lambda b: (b, 0, 0)),
    out_shape=jax.ShapeDtypeStruct((_B, _H // 8, _W), jnp.int32),
)


_sc_mesh = plsc.VectorSubcoreMesh(core_axis_name="c", subcore_axis_name="s")


@functools.partial(
    pl.kernel,
    mesh=_sc_mesh,
    compiler_params=pltpu.CompilerParams(needs_layout_passes=False),
    out_type=jax.ShapeDtypeStruct((32, 15, 16), jnp.float32),
    scratch_types=[
        pltpu.VMEM((_WORDS,), jnp.int32),
        pltpu.VMEM((9, 16), jnp.float32),
        pltpu.VMEM((15, 16), jnp.float32),
        pltpu.VMEM((16, 16), jnp.float32),
        pltpu.VMEM((16, 16), jnp.float32),
        pltpu.VMEM((16, 16), jnp.float32),
    ],
)
def _sc_moments(packed_hbm, hb_hbm, out_hbm, table_v, h_v, mom_v,
                hist_n, hist_c, hist_c2):
    cid = lax.axis_index("c")
    sid = lax.axis_index("s")
    wid = sid * 2 + cid
    batch = wid >> 1
    half = wid & 1

    pltpu.sync_copy(packed_hbm.at[batch], table_v)
    pltpu.sync_copy(hb_hbm.at[batch], h_v)

    h00 = h_v[0]
    h01 = h_v[1]
    h02 = h_v[2]
    h10 = h_v[3]
    h11 = h_v[4]
    h12 = h_v[5]
    h20 = h_v[6]
    h21 = h_v[7]
    h22 = h_v[8]

    iota16 = lax.iota(jnp.int32, 16)
    iotaf = iota16.astype(jnp.float32)
    z = jnp.zeros((16,), jnp.float32)

    accs = [z] * 15

    r_base = (half * (_H // 2)).astype(jnp.float32)
    for g in range(_GROUPS_PER_TEC):
        rvf = r_base + (g * 16) + iotaf
        bx = h01 * rvf + h02
        by = h11 * rvf + h12
        bd = h21 * rvf + h22

        def body(ci, carry, bx=bx, by=by, bd=bd):
            c3, s3, q3, c4, s4, q4, c8, s8, q8 = carry
            cfv = ci.astype(jnp.float32) + z
            rcp = 1.0 / (h20 * cfv + bd)
            xs = (h00 * cfv + bx) * rcp
            ys = (h10 * cfv + by) * rcp
            xi = (jnp.clip(xs, 0.0, float(_W - 1)) + 0.5).astype(jnp.int32)
            yi = (jnp.clip(ys, 0.0, float(_H - 1)) + 0.5).astype(jnp.int32)
            lin = (yi >> 3) * _W + xi
            shift = (yi & 7) << 2
            w = plsc.load_gather(table_v, [lin])
            code = (w >> shift) & 15
            m3 = jnp.where(code == 3, 1.0, 0.0)
            m4 = jnp.where(code == 4, 1.0, 0.0)
            m8 = jnp.where(code == 8, 1.0, 0.0)
            cf2 = cfv * cfv
            return (c3 + m3, s3 + cfv * m3, q3 + cf2 * m3,
                    c4 + m4, s4 + cfv * m4, q4 + cf2 * m4,
                    c8 + m8, s8 + cfv * m8, q8 + cf2 * m8)

        c3, s3, q3, c4, s4, q4, c8, s8, q8 = lax.fori_loop(
            0, _W, body, (z, z, z, z, z, z, z, z, z), unroll=8)

        rvf2 = rvf * rvf
        for slot, (cnt, sc_, sc2_) in enumerate(
                ((c3, s3, q3), (c4, s4, q4), (c8, s8, q8))):
            accs[5 * slot + 0] = accs[5 * slot + 0] + cnt
            accs[5 * slot + 1] = accs[5 * slot + 1] + rvf * cnt
            accs[5 * slot + 2] = accs[5 * slot + 2] + rvf2 * cnt
            accs[5 * slot + 3] = accs[5 * slot + 3] + sc_
            accs[5 * slot + 4] = accs[5 * slot + 4] + sc2_

    for i in range(15):
        mom_v[i] = accs[i]
    pltpu.sync_copy(mom_v, out_hbm.at[wid])


def _loss_body(m_ref, w_ref, ns_ref, out_ref):
    s = (m_ref[:, 0] + m_ref[:, 1]).sum(axis=2)
    lin = jnp.dot(s, w_ref[...], preferred_element_type=jnp.float32)
    nsel = jnp.dot(s, ns_ref[...], preferred_element_type=jnp.float32)
    out_ref[...] = jnp.sum(lin / jnp.maximum(1.0, nsel)).reshape(1, 1)


_loss_call = pl.pallas_call(
    _loss_body,
    out_shape=jax.ShapeDtypeStruct((1, 1), jnp.float32),
)


def kernel(input_label, h, device=0):
    lab4 = input_label.reshape(_B, _H // 8, 8, _W)
    packed = _pack_call(lab4).reshape(_B, _WORDS)
    hb = jnp.broadcast_to(
        h.astype(jnp.float32).reshape(_B, 9, 1), (_B, 9, 16))
    mom = _sc_moments(packed, hb)
    m4d = mom.reshape(_B, 2, 15, 16)
    m4d = jnp.pad(m4d, ((0, 0), (0, 0), (0, 1), (0, 0)))
    loss = _loss_call(m4d, jnp.asarray(_WMAT), jnp.asarray(_NSEL))
    return loss[0, 0]


# speedup vs baseline: 2.6502x; 1.1159x over previous
import functools

import numpy as np
import jax
import jax.numpy as jnp
from jax import lax
from jax.experimental import pallas as pl
from jax.experimental.pallas import tpu as pltpu
from jax.experimental.pallas import tpu_sc as plsc

_B, _H, _W = 16, 480, 640
_GROUPS_PER_TEC = (_H // 2) // 16
_WORDS = _H * _W // 8

_WMAT = np.zeros((16, 8), np.float32)
_NSEL = np.zeros((16, 8), np.float32)
for _slot, (_J, _Ax, _Bx, _Ay, _By) in enumerate([
    (2.0, 225.0, 50625.0, 128.0, 16384.0),
    (1.0, 0.0, 0.0, 0.0, 0.0),
    (2.0, 0.0, 0.0, 0.0, 0.0),
]):
    _WMAT[5 * _slot + 0, _slot] = _Bx + _By
    _WMAT[5 * _slot + 1, _slot] = -2.0 * _Ax
    _WMAT[5 * _slot + 2, _slot] = _J
    _WMAT[5 * _slot + 3, _slot] = -2.0 * _Ay
    _WMAT[5 * _slot + 4, _slot] = _J
    _NSEL[5 * _slot + 0, _slot] = 1.0


def _pack_body(lab_ref, out_ref):
    w = lab_ref[0, :, 0, :]
    for j in range(1, 8):
        w = w | (lab_ref[0, :, j, :] << (4 * j))
    out_ref[0] = w


_pack_call = pl.pallas_call(
    _pack_body,
    grid=(_B,),
    in_specs=[pl.BlockSpec((1, _H // 8, 8, _W), lambda b: (b, 0, 0, 0))],
    out_specs=pl.BlockSpec((1, _H // 8, _W), lambda b: (b, 0, 0)),
    out_shape=jax.ShapeDtypeStruct((_B, _H // 8, _W), jnp.int32),
)


_sc_mesh = plsc.VectorSubcoreMesh(core_axis_name="c", subcore_axis_name="s")


@functools.partial(
    pl.kernel,
    mesh=_sc_mesh,
    compiler_params=pltpu.CompilerParams(needs_layout_passes=False),
    out_type=jax.ShapeDtypeStruct((32, 15, 16), jnp.float32),
    scratch_types=[
        pltpu.VMEM((_WORDS,), jnp.int32),
        pltpu.VMEM((9, 16), jnp.float32),
        pltpu.VMEM((15, 16), jnp.float32),
        pltpu.VMEM((16, 16), jnp.float32),
        pltpu.VMEM((16, 16), jnp.float32),
        pltpu.VMEM((16, 16), jnp.float32),
    ],
)
def _sc_moments(packed_hbm, hb_hbm, out_hbm, table_v, h_v, mom_v,
                hist_n, hist_c, hist_c2):
    cid = lax.axis_index("c")
    sid = lax.axis_index("s")
    wid = sid * 2 + cid
    batch = wid >> 1
    half = wid & 1

    pltpu.sync_copy(packed_hbm.at[batch], table_v)
    pltpu.sync_copy(hb_hbm.at[batch], h_v)

    h00 = h_v[0]
    h01 = h_v[1]
    h02 = h_v[2]
    h10 = h_v[3]
    h11 = h_v[4]
    h12 = h_v[5]
    h20 = h_v[6]
    h21 = h_v[7]
    h22 = h_v[8]

    iota16 = lax.iota(jnp.int32, 16)
    iotaf = iota16.astype(jnp.float32)
    z = jnp.zeros((16,), jnp.float32)

    accs = [z] * 15

    r_base = (half * (_H // 2)).astype(jnp.float32)
    for g in range(_GROUPS_PER_TEC):
        rvf = r_base + (g * 16) + iotaf
        bx = h01 * rvf + h02
        by = h11 * rvf + h12
        bd = h21 * rvf + h22

        zi = jnp.zeros((16,), jnp.int32)

        def body(ci, carry, bx=bx, by=by, bd=bd):
            a3, q3, a4, q4, a8, q8 = carry
            cf = ci.astype(jnp.float32)
            cfv = cf + z
            combo_v = (ci | (1 << 18)) + zi
            cf2v = cf * cf + z
            rcp = 1.0 / (h20 * cfv + bd)
            xs = (h00 * cfv + bx) * rcp
            ys = (h10 * cfv + by) * rcp
            xi = (jnp.clip(xs, 0.0, float(_W - 1)) + 0.5).astype(jnp.int32)
            yi = (jnp.clip(ys, 0.0, float(_H - 1)) + 0.5).astype(jnp.int32)
            lin = (yi >> 3) * _W + xi
            shift = (yi & 7) << 2
            w = plsc.load_gather(table_v, [lin])
            code = (w >> shift) & 15
            m3 = code == 3
            m4 = code == 4
            m8 = code == 8
            return (a3 + jnp.where(m3, combo_v, zi), q3 + jnp.where(m3, cf2v, z),
                    a4 + jnp.where(m4, combo_v, zi), q4 + jnp.where(m4, cf2v, z),
                    a8 + jnp.where(m8, combo_v, zi), q8 + jnp.where(m8, cf2v, z))

        a3, q3, a4, q4, a8, q8 = lax.fori_loop(
            0, _W, body, (zi, z, zi, z, zi, z), unroll=4)

        rvf2 = rvf * rvf
        for slot, (acc_i, sc2_) in enumerate(((a3, q3), (a4, q4), (a8, q8))):
            cnt = (acc_i >> 18).astype(jnp.float32)
            sc_ = (acc_i & ((1 << 18) - 1)).astype(jnp.float32)
            accs[5 * slot + 0] = accs[5 * slot + 0] + cnt
            accs[5 * slot + 1] = accs[5 * slot + 1] + rvf * cnt
            accs[5 * slot + 2] = accs[5 * slot + 2] + rvf2 * cnt
            accs[5 * slot + 3] = accs[5 * slot + 3] + sc_
            accs[5 * slot + 4] = accs[5 * slot + 4] + sc2_

    for i in range(15):
        mom_v[i] = accs[i]
    pltpu.sync_copy(mom_v, out_hbm.at[wid])


def _loss_body(m_ref, w_ref, ns_ref, out_ref):
    s = (m_ref[:, 0] + m_ref[:, 1]).sum(axis=2)
    lin = jnp.dot(s, w_ref[...], preferred_element_type=jnp.float32)
    nsel = jnp.dot(s, ns_ref[...], preferred_element_type=jnp.float32)
    out_ref[...] = jnp.sum(lin / jnp.maximum(1.0, nsel)).reshape(1, 1)


_loss_call = pl.pallas_call(
    _loss_body,
    out_shape=jax.ShapeDtypeStruct((1, 1), jnp.float32),
)


def kernel(input_label, h, device=0):
    lab4 = input_label.reshape(_B, _H // 8, 8, _W)
    packed = _pack_call(lab4).reshape(_B, _WORDS)
    hb = jnp.broadcast_to(
        h.astype(jnp.float32).reshape(_B, 9, 1), (_B, 9, 16))
    mom = _sc_moments(packed, hb)
    m4d = mom.reshape(_B, 2, 15, 16)
    m4d = jnp.pad(m4d, ((0, 0), (0, 0), (0, 1), (0, 0)))
    loss = _loss_call(m4d, jnp.asarray(_WMAT), jnp.asarray(_NSEL))
    return loss[0, 0]


# speedup vs baseline: 2.6502x; 1.0000x over previous
import functools

import numpy as np
import jax
import jax.numpy as jnp
from jax import lax
from jax.experimental import pallas as pl
from jax.experimental.pallas import tpu as pltpu
from jax.experimental.pallas import tpu_sc as plsc

_B, _H, _W = 16, 480, 640
_GROUPS_PER_TEC = (_H // 2) // 16
_WORDS = _H * _W // 8

_WMAT = np.zeros((16, 8), np.float32)
_NSEL = np.zeros((16, 8), np.float32)
for _slot, (_J, _Ax, _Bx, _Ay, _By) in enumerate([
    (2.0, 225.0, 50625.0, 128.0, 16384.0),
    (1.0, 0.0, 0.0, 0.0, 0.0),
    (2.0, 0.0, 0.0, 0.0, 0.0),
]):
    _WMAT[5 * _slot + 0, _slot] = _Bx + _By
    _WMAT[5 * _slot + 1, _slot] = -2.0 * _Ax
    _WMAT[5 * _slot + 2, _slot] = _J
    _WMAT[5 * _slot + 3, _slot] = -2.0 * _Ay
    _WMAT[5 * _slot + 4, _slot] = _J
    _NSEL[5 * _slot + 0, _slot] = 1.0


def _pack_body(lab_ref, out_ref):
    w = lab_ref[0, :, 0, :]
    for j in range(1, 8):
        w = w | (lab_ref[0, :, j, :] << (4 * j))
    out_ref[0] = w


_pack_call = pl.pallas_call(
    _pack_body,
    grid=(_B,),
    in_specs=[pl.BlockSpec((1, _H // 8, 8, _W), lambda b: (b, 0, 0, 0))],
    out_specs=pl.BlockSpec((1, _H // 8, _W), lambda b: (b, 0, 0)),
    out_shape=jax.ShapeDtypeStruct((_B, _H // 8, _W), jnp.int32),
)


_sc_mesh = plsc.VectorSubcoreMesh(core_axis_name="c", subcore_axis_name="s")


@functools.partial(
    pl.kernel,
    mesh=_sc_mesh,
    compiler_params=pltpu.CompilerParams(needs_layout_passes=False),
    out_type=jax.ShapeDtypeStruct((32, 15, 16), jnp.float32),
    scratch_types=[
        pltpu.VMEM((_WORDS,), jnp.int32),
        pltpu.VMEM((9, 16), jnp.float32),
        pltpu.VMEM((15, 16), jnp.float32),
        pltpu.VMEM((16, 16), jnp.float32),
        pltpu.VMEM((16, 16), jnp.float32),
        pltpu.VMEM((16, 16), jnp.float32),
    ],
)
def _sc_moments(packed_hbm, hb_hbm, out_hbm, table_v, h_v, mom_v,
                hist_n, hist_c, hist_c2):
    cid = lax.axis_index("c")
    sid = lax.axis_index("s")
    wid = sid * 2 + cid
    batch = wid >> 1
    half = wid & 1

    pltpu.sync_copy(packed_hbm.at[batch], table_v)
    pltpu.sync_copy(hb_hbm.at[batch], h_v)

    h00 = h_v[0]
    h01 = h_v[1]
    h02 = h_v[2]
    h10 = h_v[3]
    h11 = h_v[4]
    h12 = h_v[5]
    h20 = h_v[6]
    h21 = h_v[7]
    h22 = h_v[8]

    iota16 = lax.iota(jnp.int32, 16)
    iotaf = iota16.astype(jnp.float32)
    z = jnp.zeros((16,), jnp.float32)

    accs = [z] * 15

    r_base = (half * (_H // 2)).astype(jnp.float32)
    for g in range(_GROUPS_PER_TEC):
        rvf = r_base + (g * 16) + iotaf
        bx = h01 * rvf + h02
        by = h11 * rvf + h12
        bd = h21 * rvf + h22

        zi = jnp.zeros((16,), jnp.int32)

        def body(ci, carry, bx=bx, by=by, bd=bd):
            a3, q3, a4, q4, a8, q8 = carry
            cf = ci.astype(jnp.float32)
            cfv = cf + z
            combo_v = (ci | (1 << 18)) + zi
            cf2v = cf * cf + z
            rcp = 1.0 / (h20 * cfv + bd)
            xs = (h00 * cfv + bx) * rcp
            ys = (h10 * cfv + by) * rcp
            xi = (jnp.clip(xs, 0.0, float(_W - 1)) + 0.5).astype(jnp.int32)
            yi = (jnp.clip(ys, 0.0, float(_H - 1)) + 0.5).astype(jnp.int32)
            lin = (yi >> 3) * _W + xi
            shift = (yi & 7) << 2
            w = plsc.load_gather(table_v, [lin])
            code = (w >> shift) & 15
            m3 = code == 3
            m4 = code == 4
            m8 = code == 8
            return (a3 + jnp.where(m3, combo_v, zi), q3 + jnp.where(m3, cf2v, z),
                    a4 + jnp.where(m4, combo_v, zi), q4 + jnp.where(m4, cf2v, z),
                    a8 + jnp.where(m8, combo_v, zi), q8 + jnp.where(m8, cf2v, z))

        a3, q3, a4, q4, a8, q8 = plsc.parallel_loop(
            0, _W, unroll=4, carry=(zi, z, zi, z, zi, z))(body)

        rvf2 = rvf * rvf
        for slot, (acc_i, sc2_) in enumerate(((a3, q3), (a4, q4), (a8, q8))):
            cnt = (acc_i >> 18).astype(jnp.float32)
            sc_ = (acc_i & ((1 << 18) - 1)).astype(jnp.float32)
            accs[5 * slot + 0] = accs[5 * slot + 0] + cnt
            accs[5 * slot + 1] = accs[5 * slot + 1] + rvf * cnt
            accs[5 * slot + 2] = accs[5 * slot + 2] + rvf2 * cnt
            accs[5 * slot + 3] = accs[5 * slot + 3] + sc_
            accs[5 * slot + 4] = accs[5 * slot + 4] + sc2_

    for i in range(15):
        mom_v[i] = accs[i]
    pltpu.sync_copy(mom_v, out_hbm.at[wid])


def _loss_body(m_ref, w_ref, ns_ref, out_ref):
    s = (m_ref[:, 0] + m_ref[:, 1]).sum(axis=2)
    lin = jnp.dot(s, w_ref[...], preferred_element_type=jnp.float32)
    nsel = jnp.dot(s, ns_ref[...], preferred_element_type=jnp.float32)
    out_ref[...] = jnp.sum(lin / jnp.maximum(1.0, nsel)).reshape(1, 1)


_loss_call = pl.pallas_call(
    _loss_body,
    out_shape=jax.ShapeDtypeStruct((1, 1), jnp.float32),
)


def kernel(input_label, h, device=0):
    lab4 = input_label.reshape(_B, _H // 8, 8, _W)
    packed = _pack_call(lab4).reshape(_B, _WORDS)
    hb = jnp.broadcast_to(
        h.astype(jnp.float32).reshape(_B, 9, 1), (_B, 9, 16))
    mom = _sc_moments(packed, hb)
    m4d = mom.reshape(_B, 2, 15, 16)
    m4d = jnp.pad(m4d, ((0, 0), (0, 0), (0, 1), (0, 0)))
    loss = _loss_call(m4d, jnp.asarray(_WMAT), jnp.asarray(_NSEL))
    return loss[0, 0]
